# Initial kernel scaffold; baseline (speedup 1.0000x reference)
import jax
import jax.numpy as jnp
from jax.experimental import pallas as pl


def kernel(raw, ys, A):
    raise NotImplementedError("write your pallas kernel here")


# speedup vs baseline: 3.3332x; 3.3332x over previous
import jax
import jax.numpy as jnp
from jax.experimental import pallas as pl
from jax.experimental.pallas import tpu as pltpu

_NK = 8
_NP = 80


def _round_bf16_bits(x):
    u = jax.lax.bitcast_convert_type(x, jnp.int32)
    odd = jax.lax.shift_right_logical(u, 16) & 1
    u = (u + 0x7FFF + odd) & jnp.int32(-65536)
    return jax.lax.bitcast_convert_type(u, jnp.float32)


def _prep_params(ys, A):
    eps = 0.0001
    neg = jnp.sum(A * (A < 0), axis=1)
    pos = jnp.sum(A * (A > 0), axis=1)
    ys_full = jnp.concatenate([neg[..., None], ys, pos[..., None]], axis=-1)
    lin = jnp.linspace(0.0, 1.0, _NK + 2)
    xs = lin[None, None, :] * (pos + eps - neg)[..., None] + neg[..., None]
    dx0 = xs[..., 1] - xs[..., 0]
    slopes = jnp.diff(ys_full, axis=-1) / dx0[..., None]
    s0 = slopes[..., 0]
    d = slopes[..., 1:] - slopes[..., :-1]
    knots = xs[..., 1:-1]
    c0 = ys_full[..., 0] - s0 * xs[..., 0] - jnp.sum(d * knots, axis=-1)
    pinv = jnp.linalg.pinv(A)
    B = A.shape[0]
    return jnp.concatenate(
        [
            _round_bf16_bits(A.reshape(B, 9)),
            _round_bf16_bits(pinv.reshape(B, 9)),
            s0,
            c0,
            knots.reshape(B, 24),
            d.reshape(B, 24),
            xs[..., 0],
            xs[..., -1],
            jnp.zeros((B, 2), jnp.float32),
        ],
        axis=-1,
    ).reshape(B, 1, _NP)


def _tc_body(params_ref, raw_ref, out_ref):
    r = _round_bf16_bits(raw_ref[0, 0])
    g = _round_bf16_bits(raw_ref[0, 1])
    b = _round_bf16_bits(raw_ref[0, 2])

    def P(k):
        return params_ref[0, 0, k]

    ests = []
    for a in range(3):
        x = r * P(a) + g * P(3 + a) + b * P(6 + a)
        f = P(21 + a) + P(18 + a) * x
        for i in range(_NK):
            f = f + P(48 + a * 8 + i) * jnp.maximum(x, P(24 + a * 8 + i))
        oob = (x < P(72 + a)) | (x > P(75 + a))
        f = jnp.where(oob, 99.0, f)
        ests.append(_round_bf16_bits(f))
    for c in range(3):
        out_ref[0, c] = (
            ests[0] * P(9 + c) + ests[1] * P(12 + c) + ests[2] * P(15 + c)
        )


@jax.jit
def kernel(raw, ys, A):
    B, C, H, W = raw.shape
    params = _prep_params(ys, A)
    HB = 256
    out = pl.pallas_call(
        _tc_body,
        grid=(B, H // HB),
        in_specs=[
            pl.BlockSpec(
                (1, 1, _NP), lambda b, h: (b, 0, 0), memory_space=pltpu.SMEM
            ),
            pl.BlockSpec((1, C, HB, W), lambda b, h: (b, 0, h, 0)),
        ],
        out_specs=pl.BlockSpec((1, C, HB, W), lambda b, h: (b, 0, h, 0)),
        out_shape=jax.ShapeDtypeStruct(raw.shape, raw.dtype),
    )(params, raw)
    return out
